# Initial kernel scaffold; baseline (speedup 1.0000x reference)
import jax
import jax.numpy as jnp
from jax.experimental import pallas as pl


def kernel(fusion_fs, lan_fs):
    raise NotImplementedError("write your pallas kernel here")


# speedup vs baseline: 1.3493x; 1.3493x over previous
import jax
import jax.numpy as jnp
from jax.experimental import pallas as pl
from jax.experimental.pallas import tpu as pltpu

B = 128
V = 196
D = 768
TA = 8
NSTEPS = B // TA


def _sim_kernel(l_ref, vis_ref, loss_ref, idx_ref, m0_ref, m1_ref, acc_ref):
    i = pl.program_id(0)
    lmat = l_ref[...]
    iota_v = jax.lax.broadcasted_iota(jnp.int32, (V, B), 0)

    m0_rows = []
    m1_rows = []
    s_sum = None
    for j in range(TA):
        v = vis_ref[j]
        st = jax.lax.dot_general(
            v, lmat, (((1,), (1,)), ((), ())),
            preferred_element_type=jnp.float32)
        m0 = jnp.max(st, axis=0, keepdims=True)
        first = jnp.min(jnp.where(st == m0, iota_v, V), axis=0, keepdims=True)
        st_masked = jnp.where(iota_v == first, -jnp.inf, st)
        m1 = jnp.max(st_masked, axis=0, keepdims=True)
        m0_rows.append(m0)
        m1_rows.append(m1)
        s_sum = st if s_sum is None else s_sum + st

    row0 = i * TA
    m0_ref[pl.ds(row0, TA), :] = jnp.concatenate(m0_rows, axis=0)
    m1_ref[pl.ds(row0, TA), :] = jnp.concatenate(m1_rows, axis=0)

    @pl.when(i == 0)
    def _():
        acc_ref[...] = s_sum

    @pl.when(i > 0)
    def _():
        acc_ref[...] += s_sum

    @pl.when(i == NSTEPS - 1)
    def _():
        acc = acc_ref[...]
        colmax = jnp.max(acc, axis=0, keepdims=True)
        idx_ref[...] = jnp.min(
            jnp.where(acc == colmax, iota_v, V), axis=0, keepdims=True)

        m0m = m0_ref[...]
        m1m = m1_ref[...]
        mcol = jnp.maximum(jnp.max(m0m, axis=0, keepdims=True),
                           jnp.max(m1m, axis=0, keepdims=True))
        e0 = jnp.exp(m0m - mcol)
        e1 = jnp.exp(m1m - mcol)
        lanes = jax.lax.broadcasted_iota(jnp.int32, (B, B), 1)
        rows = jax.lax.broadcasted_iota(jnp.int32, (B, B), 0)
        diag = lanes == rows
        z = (jnp.sum(e0, axis=0) + jnp.sum(e1, axis=0)
             - jnp.sum(jnp.where(diag, e1, 0.0), axis=0))
        logz = jnp.log(z) + mcol[0]
        diag0 = jnp.sum(jnp.where(diag, m0m, 0.0), axis=0)
        loss_ref[...] = (jnp.sum(logz - diag0) * (1.0 / B)).reshape(1, 1)


def _gather_kernel(idx_ref, vis_ref, emb_ref):
    del idx_ref
    emb_ref[...] = vis_ref[...]


@jax.jit
def kernel(fusion_fs, lan_fs):
    vis = fusion_fs
    lmat = lan_fs.reshape(B, D)

    loss2d, idx2d = pl.pallas_call(
        _sim_kernel,
        grid=(NSTEPS,),
        in_specs=[
            pl.BlockSpec((B, D), lambda i: (0, 0)),
            pl.BlockSpec((TA, V, D), lambda i: (i, 0, 0)),
        ],
        out_specs=[
            pl.BlockSpec((1, 1), lambda i: (0, 0)),
            pl.BlockSpec((1, B), lambda i: (0, 0)),
        ],
        out_shape=[
            jax.ShapeDtypeStruct((1, 1), jnp.float32),
            jax.ShapeDtypeStruct((1, B), jnp.int32),
        ],
        scratch_shapes=[
            pltpu.VMEM((B, B), jnp.float32),
            pltpu.VMEM((B, B), jnp.float32),
            pltpu.VMEM((V, B), jnp.float32),
        ],
    )(lmat, vis)

    vis_flat = vis.reshape(B * V, 1, D)
    emb = pl.pallas_call(
        _gather_kernel,
        grid_spec=pltpu.PrefetchScalarGridSpec(
            num_scalar_prefetch=1,
            grid=(B,),
            in_specs=[
                pl.BlockSpec((1, 1, D), lambda i, idx: (i * V + idx[i], 0, 0)),
            ],
            out_specs=pl.BlockSpec((1, 1, D), lambda i, idx: (i, 0, 0)),
        ),
        out_shape=jax.ShapeDtypeStruct((B, 1, D), jnp.float32),
    )(idx2d[0], vis_flat)

    return loss2d[0, 0], emb


# speedup vs baseline: 1.5401x; 1.1414x over previous
import jax
import jax.numpy as jnp
from jax.experimental import pallas as pl
from jax.experimental.pallas import tpu as pltpu

B = 128
V = 196
D = 768
TA = 8
NSTEPS = B // TA


def _sim_kernel(l_ref, vis_ref, loss_ref, idx_ref, m0_ref, m1_ref, acc_ref):
    i = pl.program_id(0)
    lmat = l_ref[...]
    iota_v = jax.lax.broadcasted_iota(jnp.int32, (V, B), 0)

    m0_rows = []
    m1_rows = []
    s_sum = None
    for j in range(TA):
        v = vis_ref[j]
        st = jax.lax.dot_general(
            v, lmat, (((1,), (1,)), ((), ())),
            preferred_element_type=jnp.float32)
        m0 = jnp.max(st, axis=0, keepdims=True)
        first = jnp.min(jnp.where(st == m0, iota_v, V), axis=0, keepdims=True)
        st_masked = jnp.where(iota_v == first, -jnp.inf, st)
        m1 = jnp.max(st_masked, axis=0, keepdims=True)
        m0_rows.append(m0)
        m1_rows.append(m1)
        s_sum = st if s_sum is None else s_sum + st

    row0 = i * TA
    m0_ref[pl.ds(row0, TA), :] = jnp.concatenate(m0_rows, axis=0)
    m1_ref[pl.ds(row0, TA), :] = jnp.concatenate(m1_rows, axis=0)

    @pl.when(i == 0)
    def _():
        acc_ref[...] = s_sum

    @pl.when(i > 0)
    def _():
        acc_ref[...] += s_sum

    @pl.when(i == NSTEPS - 1)
    def _():
        acc = acc_ref[...]
        colmax = jnp.max(acc, axis=0, keepdims=True)
        idx_ref[...] = jnp.min(
            jnp.where(acc == colmax, iota_v, V), axis=0, keepdims=True)

        m0m = m0_ref[...]
        m1m = m1_ref[...]
        mcol = jnp.maximum(jnp.max(m0m, axis=0, keepdims=True),
                           jnp.max(m1m, axis=0, keepdims=True))
        e0 = jnp.exp(m0m - mcol)
        e1 = jnp.exp(m1m - mcol)
        lanes = jax.lax.broadcasted_iota(jnp.int32, (B, B), 1)
        rows = jax.lax.broadcasted_iota(jnp.int32, (B, B), 0)
        diag = lanes == rows
        z = (jnp.sum(e0, axis=0) + jnp.sum(e1, axis=0)
             - jnp.sum(jnp.where(diag, e1, 0.0), axis=0))
        logz = jnp.log(z) + mcol[0]
        diag0 = jnp.sum(jnp.where(diag, m0m, 0.0), axis=0)
        loss_ref[...] = (jnp.sum(logz - diag0) * (1.0 / B)).reshape(1, 1)


GROWS = 8


def _gather_kernel(idx_ref, *refs):
    del idx_ref
    emb_ref = refs[-1]
    for k in range(GROWS):
        emb_ref[k] = refs[k][0]


@jax.jit
def kernel(fusion_fs, lan_fs):
    vis = fusion_fs
    lmat = lan_fs.reshape(B, D)

    loss2d, idx2d = pl.pallas_call(
        _sim_kernel,
        grid=(NSTEPS,),
        in_specs=[
            pl.BlockSpec((B, D), lambda i: (0, 0)),
            pl.BlockSpec((TA, V, D), lambda i: (i, 0, 0)),
        ],
        out_specs=[
            pl.BlockSpec((1, 1), lambda i: (0, 0)),
            pl.BlockSpec((1, B), lambda i: (0, 0)),
        ],
        out_shape=[
            jax.ShapeDtypeStruct((1, 1), jnp.float32),
            jax.ShapeDtypeStruct((1, B), jnp.int32),
        ],
        scratch_shapes=[
            pltpu.VMEM((B, B), jnp.float32),
            pltpu.VMEM((B, B), jnp.float32),
            pltpu.VMEM((V, B), jnp.float32),
        ],
    )(lmat, vis)

    vis_flat = vis.reshape(B * V, 1, D)

    def _row_spec(k):
        return pl.BlockSpec(
            (1, 1, D),
            lambda i, idx, k=k: ((i * GROWS + k) * V + idx[i * GROWS + k], 0, 0))

    emb = pl.pallas_call(
        _gather_kernel,
        grid_spec=pltpu.PrefetchScalarGridSpec(
            num_scalar_prefetch=1,
            grid=(B // GROWS,),
            in_specs=[_row_spec(k) for k in range(GROWS)],
            out_specs=pl.BlockSpec((GROWS, 1, D), lambda i, idx: (i, 0, 0)),
        ),
        out_shape=jax.ShapeDtypeStruct((B, 1, D), jnp.float32),
    )(idx2d[0], *([vis_flat] * GROWS))

    return loss2d[0, 0], emb


# speedup vs baseline: 1.5633x; 1.0151x over previous
import jax
import jax.numpy as jnp
from jax.experimental import pallas as pl
from jax.experimental.pallas import tpu as pltpu

B = 128
V = 196
D = 768
TA = 8
NSTEPS = B // TA


def _sim_kernel(l_ref, vis_ref, loss_ref, idx_ref, m0_ref, m1_ref, vsum_ref):
    i = pl.program_id(0)
    lmat = l_ref[...]
    iota_v = jax.lax.broadcasted_iota(jnp.int32, (V, B), 0)

    m0_rows = []
    m1_rows = []
    block_sum = None
    for j in range(TA):
        v = vis_ref[j]
        block_sum = v if block_sum is None else block_sum + v
        st = jax.lax.dot_general(
            v, lmat, (((1,), (1,)), ((), ())),
            preferred_element_type=jnp.float32)
        m0 = jnp.max(st, axis=0, keepdims=True)
        first = jnp.min(jnp.where(st == m0, iota_v, V), axis=0, keepdims=True)
        st_masked = jnp.where(iota_v == first, -jnp.inf, st)
        m1 = jnp.max(st_masked, axis=0, keepdims=True)
        m0_rows.append(m0)
        m1_rows.append(m1)

    row0 = i * TA
    m0_ref[pl.ds(row0, TA), :] = jnp.concatenate(m0_rows, axis=0)
    m1_ref[pl.ds(row0, TA), :] = jnp.concatenate(m1_rows, axis=0)

    @pl.when(i == 0)
    def _():
        vsum_ref[...] = block_sum

    @pl.when(i > 0)
    def _():
        vsum_ref[...] += block_sum

    @pl.when(i == NSTEPS - 1)
    def _():
        anchor = jax.lax.dot_general(
            vsum_ref[...], lmat, (((1,), (1,)), ((), ())),
            preferred_element_type=jnp.float32)
        colmax = jnp.max(anchor, axis=0, keepdims=True)
        idx_ref[...] = jnp.min(
            jnp.where(anchor == colmax, iota_v, V), axis=0, keepdims=True)

        m0m = m0_ref[...]
        m1m = m1_ref[...]
        mcol = jnp.maximum(jnp.max(m0m, axis=0, keepdims=True),
                           jnp.max(m1m, axis=0, keepdims=True))
        e0 = jnp.exp(m0m - mcol)
        e1 = jnp.exp(m1m - mcol)
        lanes = jax.lax.broadcasted_iota(jnp.int32, (B, B), 1)
        rows = jax.lax.broadcasted_iota(jnp.int32, (B, B), 0)
        diag = lanes == rows
        z = (jnp.sum(e0, axis=0) + jnp.sum(e1, axis=0)
             - jnp.sum(jnp.where(diag, e1, 0.0), axis=0))
        logz = jnp.log(z) + mcol[0]
        diag0 = jnp.sum(jnp.where(diag, m0m, 0.0), axis=0)
        loss_ref[...] = (jnp.sum(logz - diag0) * (1.0 / B)).reshape(1, 1)


def _gather_kernel(idx_ref, vis_ref, emb_ref, sem):
    copies = []
    for b in range(B):
        flat = b * V + idx_ref[b]
        c = pltpu.make_async_copy(
            vis_ref.at[pl.ds(flat, 1)], emb_ref.at[pl.ds(b, 1)], sem)
        c.start()
        copies.append(c)
    for c in copies:
        c.wait()


@jax.jit
def kernel(fusion_fs, lan_fs):
    vis = fusion_fs
    lmat = lan_fs.reshape(B, D)

    loss2d, idx2d = pl.pallas_call(
        _sim_kernel,
        grid=(NSTEPS,),
        in_specs=[
            pl.BlockSpec((B, D), lambda i: (0, 0)),
            pl.BlockSpec((TA, V, D), lambda i: (i, 0, 0)),
        ],
        out_specs=[
            pl.BlockSpec((1, 1), lambda i: (0, 0)),
            pl.BlockSpec((1, B), lambda i: (0, 0)),
        ],
        out_shape=[
            jax.ShapeDtypeStruct((1, 1), jnp.float32),
            jax.ShapeDtypeStruct((1, B), jnp.int32),
        ],
        scratch_shapes=[
            pltpu.VMEM((B, B), jnp.float32),
            pltpu.VMEM((B, B), jnp.float32),
            pltpu.VMEM((V, D), jnp.float32),
        ],
    )(lmat, vis)

    vis_flat = vis.reshape(B * V, 1, D)
    emb = pl.pallas_call(
        _gather_kernel,
        grid_spec=pltpu.PrefetchScalarGridSpec(
            num_scalar_prefetch=1,
            grid=(1,),
            in_specs=[pl.BlockSpec(memory_space=pl.ANY)],
            out_specs=pl.BlockSpec((B, 1, D), lambda i, idx: (0, 0, 0)),
            scratch_shapes=[pltpu.SemaphoreType.DMA],
        ),
        out_shape=jax.ShapeDtypeStruct((B, 1, D), jnp.float32),
    )(idx2d[0], vis_flat)

    return loss2d[0, 0], emb


# speedup vs baseline: 5.0342x; 3.2204x over previous
import jax
import jax.numpy as jnp
from jax.experimental import pallas as pl
from jax.experimental.pallas import tpu as pltpu

B = 128
V = 196
D = 768
TA = 8
NSTEPS = B // TA


def _sim_kernel(l_ref, vis_ref, loss_ref, idx_ref, m0_ref, m1_ref, vsum_ref):
    i = pl.program_id(0)
    lmat = l_ref[...]
    iota_v = jax.lax.broadcasted_iota(jnp.int32, (V, B), 0)

    m0_rows = []
    m1_rows = []
    block_sum = None
    for j in range(TA):
        v = vis_ref[j]
        block_sum = v if block_sum is None else block_sum + v
        st = jax.lax.dot_general(
            v, lmat, (((1,), (1,)), ((), ())),
            preferred_element_type=jnp.float32)
        m0 = jnp.max(st, axis=0, keepdims=True)
        first = jnp.min(jnp.where(st == m0, iota_v, V), axis=0, keepdims=True)
        st_masked = jnp.where(iota_v == first, -jnp.inf, st)
        m1 = jnp.max(st_masked, axis=0, keepdims=True)
        m0_rows.append(m0)
        m1_rows.append(m1)

    row0 = i * TA
    m0_ref[pl.ds(row0, TA), :] = jnp.concatenate(m0_rows, axis=0)
    m1_ref[pl.ds(row0, TA), :] = jnp.concatenate(m1_rows, axis=0)

    @pl.when(i == 0)
    def _():
        vsum_ref[...] = block_sum

    @pl.when(i > 0)
    def _():
        vsum_ref[...] += block_sum

    @pl.when(i == NSTEPS - 1)
    def _():
        anchor = jax.lax.dot_general(
            vsum_ref[...], lmat, (((1,), (1,)), ((), ())),
            preferred_element_type=jnp.float32)
        colmax = jnp.max(anchor, axis=0, keepdims=True)
        idx_ref[...] = jnp.min(
            jnp.where(anchor == colmax, iota_v, V), axis=0, keepdims=True)

        m0m = m0_ref[...]
        m1m = m1_ref[...]
        mcol = jnp.maximum(jnp.max(m0m, axis=0, keepdims=True),
                           jnp.max(m1m, axis=0, keepdims=True))
        e0 = jnp.exp(m0m - mcol)
        e1 = jnp.exp(m1m - mcol)
        lanes = jax.lax.broadcasted_iota(jnp.int32, (B, B), 1)
        rows = jax.lax.broadcasted_iota(jnp.int32, (B, B), 0)
        diag = lanes == rows
        z = (jnp.sum(e0, axis=0) + jnp.sum(e1, axis=0)
             - jnp.sum(jnp.where(diag, e1, 0.0), axis=0))
        logz = jnp.log(z) + mcol[0]
        diag0 = jnp.sum(jnp.where(diag, m0m, 0.0), axis=0)
        loss_ref[...] = (jnp.sum(logz - diag0) * (1.0 / B)).reshape(1, 1)


def _gather_kernel(idx_ref, vis_ref, emb_ref, sem):
    copies = []
    for b in range(B):
        c = pltpu.make_async_copy(
            vis_ref.at[b, pl.ds(idx_ref[b], 1), :], emb_ref.at[b], sem)
        c.start()
        copies.append(c)
    for c in copies:
        c.wait()


@jax.jit
def kernel(fusion_fs, lan_fs):
    vis = fusion_fs
    lmat = lan_fs.reshape(B, D)

    loss2d, idx2d = pl.pallas_call(
        _sim_kernel,
        grid=(NSTEPS,),
        in_specs=[
            pl.BlockSpec((B, D), lambda i: (0, 0)),
            pl.BlockSpec((TA, V, D), lambda i: (i, 0, 0)),
        ],
        out_specs=[
            pl.BlockSpec((1, 1), lambda i: (0, 0)),
            pl.BlockSpec((1, B), lambda i: (0, 0)),
        ],
        out_shape=[
            jax.ShapeDtypeStruct((1, 1), jnp.float32),
            jax.ShapeDtypeStruct((1, B), jnp.int32),
        ],
        scratch_shapes=[
            pltpu.VMEM((B, B), jnp.float32),
            pltpu.VMEM((B, B), jnp.float32),
            pltpu.VMEM((V, D), jnp.float32),
        ],
    )(lmat, vis)

    emb = pl.pallas_call(
        _gather_kernel,
        grid_spec=pltpu.PrefetchScalarGridSpec(
            num_scalar_prefetch=1,
            grid=(1,),
            in_specs=[pl.BlockSpec(memory_space=pl.ANY)],
            out_specs=pl.BlockSpec((B, 1, D), lambda i, idx: (0, 0, 0)),
            scratch_shapes=[pltpu.SemaphoreType.DMA],
        ),
        out_shape=jax.ShapeDtypeStruct((B, 1, D), jnp.float32),
    )(idx2d[0], vis)

    return loss2d[0, 0], emb


# speedup vs baseline: 14.9541x; 2.9705x over previous
import jax
import jax.numpy as jnp
from jax.experimental import pallas as pl
from jax.experimental.pallas import tpu as pltpu

B = 128
V = 196
D = 768
TV = 32
NSTEPS = 7
VP = NSTEPS * TV

NEG_INF = float("-inf")


def _sim_kernel(l_ref, vis_ref, loss_ref, idx_ref, m0_ref, m1_ref, vsum_ref):
    i = pl.program_id(0)
    lmat = l_ref[...]

    @pl.when(i == 0)
    def _():
        m0_ref[...] = jnp.full((B, B), NEG_INF, jnp.float32)
        m1_ref[...] = jnp.full((B, B), NEG_INF, jnp.float32)

    vsum_ref[pl.ds(i * TV, TV), :] = jnp.sum(vis_ref[...], axis=1)

    m0 = m0_ref[...]
    m1 = m1_ref[...]
    for j in range(TV):
        s = jax.lax.dot_general(
            vis_ref[j], lmat, (((1,), (1,)), ((), ())),
            preferred_element_type=jnp.float32)
        s = jnp.where(i * TV + j < V, s, NEG_INF)
        hi = jnp.maximum(m0, s)
        lo = jnp.minimum(m0, s)
        m0 = hi
        m1 = jnp.maximum(m1, lo)
    m0_ref[...] = m0
    m1_ref[...] = m1

    @pl.when(i == NSTEPS - 1)
    def _():
        anchor = jax.lax.dot_general(
            vsum_ref[...], lmat, (((1,), (1,)), ((), ())),
            preferred_element_type=jnp.float32)
        iota_v = jax.lax.broadcasted_iota(jnp.int32, (VP, B), 0)
        anchor = jnp.where(iota_v < V, anchor, NEG_INF)
        colmax = jnp.max(anchor, axis=0, keepdims=True)
        idx_ref[...] = jnp.min(
            jnp.where(anchor == colmax, iota_v, V), axis=0, keepdims=True)

        m0m = m0_ref[...]
        m1m = m1_ref[...]
        mcol = jnp.maximum(jnp.max(m0m, axis=0, keepdims=True),
                           jnp.max(m1m, axis=0, keepdims=True))
        e0 = jnp.exp(m0m - mcol)
        e1 = jnp.exp(m1m - mcol)
        lanes = jax.lax.broadcasted_iota(jnp.int32, (B, B), 1)
        rows = jax.lax.broadcasted_iota(jnp.int32, (B, B), 0)
        diag = lanes == rows
        z = (jnp.sum(e0, axis=0) + jnp.sum(e1, axis=0)
             - jnp.sum(jnp.where(diag, e1, 0.0), axis=0))
        logz = jnp.log(z) + mcol[0]
        diag0 = jnp.sum(jnp.where(diag, m0m, 0.0), axis=0)
        loss_ref[...] = (jnp.sum(logz - diag0) * (1.0 / B)).reshape(1, 1)


def _gather_kernel(idx_ref, vis_ref, emb_ref, sem):
    copies = []
    for b in range(B):
        c = pltpu.make_async_copy(
            vis_ref.at[pl.ds(idx_ref[b], 1), b, :], emb_ref.at[b], sem)
        c.start()
        copies.append(c)
    for c in copies:
        c.wait()


@jax.jit
def kernel(fusion_fs, lan_fs):
    vis_t = jnp.transpose(fusion_fs, (1, 0, 2))
    lmat = lan_fs.reshape(B, D)

    loss2d, idx2d = pl.pallas_call(
        _sim_kernel,
        grid=(NSTEPS,),
        in_specs=[
            pl.BlockSpec((B, D), lambda i: (0, 0)),
            pl.BlockSpec((TV, B, D), lambda i: (i, 0, 0)),
        ],
        out_specs=[
            pl.BlockSpec((1, 1), lambda i: (0, 0)),
            pl.BlockSpec((1, B), lambda i: (0, 0)),
        ],
        out_shape=[
            jax.ShapeDtypeStruct((1, 1), jnp.float32),
            jax.ShapeDtypeStruct((1, B), jnp.int32),
        ],
        scratch_shapes=[
            pltpu.VMEM((B, B), jnp.float32),
            pltpu.VMEM((B, B), jnp.float32),
            pltpu.VMEM((VP, D), jnp.float32),
        ],
    )(lmat, vis_t)

    emb = pl.pallas_call(
        _gather_kernel,
        grid_spec=pltpu.PrefetchScalarGridSpec(
            num_scalar_prefetch=1,
            grid=(1,),
            in_specs=[pl.BlockSpec(memory_space=pl.ANY)],
            out_specs=pl.BlockSpec((B, 1, D), lambda i, idx: (0, 0, 0)),
            scratch_shapes=[pltpu.SemaphoreType.DMA],
        ),
        out_shape=jax.ShapeDtypeStruct((B, 1, D), jnp.float32),
    )(idx2d[0], vis_t)

    return loss2d[0, 0], emb
